# baseline (speedup 1.0000x reference)
import functools

import jax
import jax.numpy as jnp
from jax import lax
from jax.experimental import pallas as pl
from jax.experimental.pallas import tpu as pltpu
from jax.experimental.pallas import tpu_sc as plsc

N = 10000
E = 160000
D = 128
H = 4
DSH = 4
DLEN = 32
DH = D * H

NC = 2
NS = 16
NW = NC * NS
EPT = E // NW
CH = 128
NFULL = EPT // CH
TOFF = EPT - CH
TDUP = NFULL * CH - TOFF
NHALF = N // NC
NPAD = 5008
EPS = E // NS
SFULL = EPS // CH
STOFF = EPS - CH
STDUP = SFULL * CH - STOFF
ZR2 = NPAD // NS

_f32 = jnp.float32


def _ln(x, eps=1e-6):
    mu = jnp.mean(x, axis=-1, keepdims=True)
    var = jnp.var(x, axis=-1, keepdims=True)
    return (x - mu) / jnp.sqrt(var + eps)


def _silu(x):
    return x * jax.nn.sigmoid(x)


_BN = 2000


def _pre_body(node_ref, wsrc_ref, wdst_ref, p_ref, q_ref):
    x = node_ref[...]
    p_ref[...] = jnp.dot(x, wsrc_ref[...], preferred_element_type=_f32)
    q_ref[...] = jnp.dot(x, wdst_ref[...], preferred_element_type=_f32)


def _pre_call(node_in, wsrc, wdst):
    g = N // _BN
    return pl.pallas_call(
        _pre_body,
        grid=(g,),
        in_specs=[
            pl.BlockSpec((_BN, D), lambda i: (i, 0)),
            pl.BlockSpec((D, D), lambda i: (0, 0)),
            pl.BlockSpec((D, D), lambda i: (0, 0)),
        ],
        out_specs=[
            pl.BlockSpec((_BN, D), lambda i: (i, 0)),
            pl.BlockSpec((_BN, D), lambda i: (i, 0)),
        ],
        out_shape=[
            jax.ShapeDtypeStruct((N, D), _f32),
            jax.ShapeDtypeStruct((N, D), _f32),
        ],
    )(node_in, wsrc, wdst)


_BA = 8000


def _alpha_body(elen_ref, sh_ref, wr1_ref, br1_ref, wr2_ref, br2_ref,
                wa1_ref, ba1_ref, wa2_ref, ba2_ref, wa3_ref, ba3_ref,
                alpha_ref, shw_ref, gmax_ref):
    el = elen_ref[...]
    r = jnp.dot(_silu(jnp.dot(el, wr1_ref[...], preferred_element_type=_f32)
                      + br1_ref[...]),
                wr2_ref[...], preferred_element_type=_f32) + br2_ref[...]
    shw_ref[...] = sh_ref[...] * r
    h1 = _silu(_ln(jnp.dot(el, wa1_ref[...], preferred_element_type=_f32)
                   + ba1_ref[...]))
    h2 = _silu(_ln(jnp.dot(h1, wa2_ref[...], preferred_element_type=_f32)
                   + ba2_ref[...]))
    alpha16 = jnp.dot(h2, wa3_ref[...], preferred_element_type=_f32) + ba3_ref[...]
    alpha_ref[...] = alpha16
    m = jnp.max(alpha16)
    i = pl.program_id(0)

    @pl.when(i == 0)
    def _():
        gmax_ref[0, 0] = m

    @pl.when(i > 0)
    def _():
        gmax_ref[0, 0] = jnp.maximum(gmax_ref[0, 0], m)


def _alpha_call(elen, edge_sh, wr1, br1, wr2, br2, wa1, ba1, wa2, ba2,
                wa3p, ba3p):
    g = E // _BA
    full = lambda r, c: pl.BlockSpec((r, c), lambda i: (0, 0))
    return pl.pallas_call(
        _alpha_body,
        grid=(g,),
        in_specs=[
            pl.BlockSpec((_BA, DLEN), lambda i: (i, 0)),
            pl.BlockSpec((_BA, DSH), lambda i: (i, 0)),
            full(DLEN, 64), full(1, 64), full(64, DSH), full(1, DSH),
            full(DLEN, 64), full(1, 64), full(64, 64), full(1, 64),
            full(64, 16), full(1, 16),
        ],
        out_specs=[
            pl.BlockSpec((_BA, 16), lambda i: (i, 0)),
            pl.BlockSpec((_BA, DSH), lambda i: (i, 0)),
            pl.BlockSpec(memory_space=pltpu.SMEM),
        ],
        out_shape=[
            jax.ShapeDtypeStruct((E, 16), _f32),
            jax.ShapeDtypeStruct((E, DSH), _f32),
            jax.ShapeDtypeStruct((1, 1), _f32),
        ],
    )(elen, edge_sh, wr1, br1, wr2, br2, wa1, ba1, wa2, ba2, wa3p, ba3p)


def _sc_mesh():
    return plsc.VectorSubcoreMesh(core_axis_name="c", subcore_axis_name="s",
                                  num_cores=NC, num_subcores=NS)


def _ssum_call(alpha16, gm16, dst):
    @functools.partial(
        pl.kernel, mesh=_sc_mesh(),
        out_type=jax.ShapeDtypeStruct((N, 16), _f32),
        compiler_params=pltpu.CompilerParams(use_tc_tiling_on_sc=False),
        scratch_types=[
            pltpu.VMEM_SHARED((NPAD, 16), _f32),
            pltpu.VMEM((ZR2, 16), _f32),
            pltpu.VMEM((CH, 16), _f32),
            pltpu.VMEM((CH,), jnp.int32),
            pltpu.VMEM((16,), _f32),
        ],
    )
    def k(alpha_hbm, gm_hbm, dst_hbm, out_hbm, shared, zbuf, ebuf, idx, gmv):
        cid = lax.axis_index("c")
        sid = lax.axis_index("s")
        base = sid * EPS
        lo = cid * NHALF

        @pl.loop(0, ZR2, unroll=8)
        def _(r):
            zbuf[r, :] = jnp.zeros((16,), _f32)

        pltpu.sync_copy(zbuf, shared.at[pl.ds(sid * ZR2, ZR2)])
        pltpu.sync_copy(gm_hbm, gmv)
        plsc.subcore_barrier()
        g = gmv[...]

        def chunk(off, ndup):
            pltpu.sync_copy(dst_hbm.at[pl.ds(off, CH)], idx)
            pltpu.sync_copy(alpha_hbm.at[pl.ds(off, CH)], ebuf)

            @pl.loop(0, CH // 16, unroll=8)
            def _(t):
                sl = pl.ds(t * 16, 16)
                dv = idx[sl] - lo
                inr = (dv >= 0) & (dv < NHALF)
                idx[sl] = jnp.where(inr, dv, NHALF)

            @pl.loop(0, CH, unroll=8)
            def _(r):
                ebuf[r, :] = jnp.exp(ebuf[r, :] - g)

            if ndup:
                @pl.loop(0, ndup)
                def _(r):
                    ebuf[r, :] = jnp.zeros((16,), _f32)

            pltpu.sync_copy(ebuf, shared.at[idx], add=True)

        @pl.loop(0, SFULL)
        def _(j):
            chunk(pl.multiple_of(base + j * CH, 8), 0)

        chunk(base + STOFF, STDUP)
        plsc.subcore_barrier()

        @pl.when(sid == 0)
        def _():
            pltpu.sync_copy(shared.at[pl.ds(0, NHALF)],
                            out_hbm.at[pl.ds(lo, NHALF)])

    return k(alpha16, gm16, dst)


def _gather_call(p_tab, q_tab, src, dst):
    @functools.partial(
        pl.kernel, mesh=_sc_mesh(),
        out_type=jax.ShapeDtypeStruct((E, D), _f32),
        scratch_types=[
            pltpu.VMEM((CH,), jnp.int32),
            pltpu.VMEM((CH,), jnp.int32),
            pltpu.VMEM((CH, D), _f32),
            pltpu.VMEM((CH, D), _f32),
            pltpu.VMEM((CH,), jnp.int32),
            pltpu.VMEM((CH,), jnp.int32),
            pltpu.VMEM((CH, D), _f32),
            pltpu.VMEM((CH, D), _f32),
            pltpu.SemaphoreType.DMA,
            pltpu.SemaphoreType.DMA,
            pltpu.SemaphoreType.DMA,
            pltpu.SemaphoreType.DMA,
            pltpu.SemaphoreType.DMA,
            pltpu.SemaphoreType.DMA,
        ],
    )
    def k(p_hbm, q_hbm, src_hbm, dst_hbm, m0_hbm,
          idxs0, idxd0, bufp0, bufq0, idxs1, idxd1, bufp1, bufq1,
          si0, si1, sg0, sg1, sw0, sw1):
        cid = lax.axis_index("c")
        sid = lax.axis_index("s")
        wid = sid * NC + cid
        base = wid * EPT
        sets = ((idxs0, idxd0, bufp0, bufq0, si0, sg0, sw0),
                (idxs1, idxd1, bufp1, bufq1, si1, sg1, sw1))

        def off_of(j):
            return pl.multiple_of(
                jnp.where(j == NFULL, base + TOFF, base + j * CH), 8)

        @pl.loop(0, (NFULL + 1) // 2)
        def _(t):
            descs = []
            for u, (ixs, ixd, bp, bq, si, sg, sw) in enumerate(sets):
                off = off_of(2 * t + u)
                descs.append((
                    pltpu.async_copy(src_hbm.at[pl.ds(off, CH)], ixs, si),
                    pltpu.async_copy(dst_hbm.at[pl.ds(off, CH)], ixd, si),
                    off))
            gds = []
            for u, (ixs, ixd, bp, bq, si, sg, sw) in enumerate(sets):
                descs[u][0].wait()
                descs[u][1].wait()
                gds.append((pltpu.async_copy(p_hbm.at[ixs], bp, sg),
                            pltpu.async_copy(q_hbm.at[ixd], bq, sg)))
            wbs = []
            for u, (ixs, ixd, bp, bq, si, sg, sw) in enumerate(sets):
                gds[u][0].wait()
                gds[u][1].wait()

                @pl.loop(0, CH, unroll=4)
                def _(r):
                    for k8 in range(D // 16):
                        sl = pl.ds(k8 * 16, 16)
                        bp[r, sl] = bp[r, sl] + bq[r, sl]

                wbs.append(pltpu.async_copy(
                    bp, m0_hbm.at[pl.ds(descs[u][2], CH)], sw))
            for u in range(2):
                wbs[u].wait()

    return k(p_tab, q_tab, src, dst)


def _sgather_call(ssum, dst):
    @functools.partial(
        pl.kernel, mesh=_sc_mesh(),
        out_type=jax.ShapeDtypeStruct((E, 16), _f32),
        compiler_params=pltpu.CompilerParams(use_tc_tiling_on_sc=False),
        scratch_types=[
            pltpu.VMEM((CH,), jnp.int32),
            pltpu.VMEM((CH, 16), _f32),
        ],
    )
    def k(ss_hbm, dst_hbm, sg_hbm, idxd, bufa):
        cid = lax.axis_index("c")
        sid = lax.axis_index("s")
        wid = sid * NC + cid
        base = wid * EPT

        def chunk(off):
            pltpu.sync_copy(dst_hbm.at[pl.ds(off, CH)], idxd)
            pltpu.sync_copy(ss_hbm.at[idxd], bufa)
            pltpu.sync_copy(bufa, sg_hbm.at[pl.ds(off, CH)])

        @pl.loop(0, NFULL)
        def _(j):
            chunk(pl.multiple_of(base + j * CH, 8))

        chunk(base + TOFF)

    return k(ssum, dst)


_BC = 2000


def _main_body(m0_ref, elen_ref, shw_ref, alpha_ref, sg_ref, gmax_ref,
               wlen_ref, wtp_ref, we_ref, ws1_ref, bs1_ref, ws2_ref,
               bs2_ref, ws3_ref, bs3_ref, wout_ref, es_ref, y_ref):
    el = elen_ref[...]
    msg = m0_ref[...] + jnp.dot(el, wlen_ref[...], preferred_element_type=_f32)
    shw = shw_ref[...].astype(jnp.bfloat16)
    msg_b = msg.astype(jnp.bfloat16)
    feat = jnp.concatenate(
        [msg_b * shw[:, s:s + 1] for s in range(DSH)], axis=1)
    value = jnp.dot(feat, wtp_ref[...],
                    preferred_element_type=_f32).astype(jnp.bfloat16)
    alpha16 = alpha_ref[...]
    es = None
    for h in range(H):
        t = jnp.dot(value[:, h * D:(h + 1) * D], we_ref[h],
                    preferred_element_type=_f32) * alpha16[:, h:h + 1]
        es = t if es is None else es + t
    es = _silu(_ln(jnp.dot(es, ws1_ref[...], preferred_element_type=_f32)
                   + bs1_ref[...]))
    es = _silu(_ln(jnp.dot(es, ws2_ref[...], preferred_element_type=_f32)
                   + bs2_ref[...]))
    es_ref[...] = jnp.dot(es, ws3_ref[...], preferred_element_type=_f32) \
        + bs3_ref[...]
    exa = jnp.exp(alpha16 - gmax_ref[0, 0])
    a = exa / (sg_ref[...] + 1e-16)
    y = None
    for h in range(H):
        t = jnp.dot(value[:, h * D:(h + 1) * D], wout_ref[h],
                    preferred_element_type=_f32) * a[:, h:h + 1]
        y = t if y is None else y + t
    y_ref[...] = y


def _main_call(m0, elen, shw, alpha16, sg, gmax, wlen, wtp3, we3,
               ws1, bs1, ws2, bs2, ws3, bs3, wout3):
    g = E // _BC
    full = lambda *s: pl.BlockSpec(s, lambda i: (0,) * len(s))
    return pl.pallas_call(
        _main_body,
        grid=(g,),
        in_specs=[
            pl.BlockSpec((_BC, D), lambda i: (i, 0)),
            pl.BlockSpec((_BC, DLEN), lambda i: (i, 0)),
            pl.BlockSpec((_BC, DSH), lambda i: (i, 0)),
            pl.BlockSpec((_BC, 16), lambda i: (i, 0)),
            pl.BlockSpec((_BC, 16), lambda i: (i, 0)),
            pl.BlockSpec(memory_space=pltpu.SMEM),
            full(DLEN, D),
            full(DSH * D, DH),
            full(H, D, 64),
            full(64, 64), full(1, 64),
            full(64, 64), full(1, 64),
            full(64, 32), full(1, 32),
            full(H, D, D),
        ],
        out_specs=[
            pl.BlockSpec((_BC, 32), lambda i: (i, 0)),
            pl.BlockSpec((_BC, D), lambda i: (i, 0)),
        ],
        out_shape=[
            jax.ShapeDtypeStruct((E, 32), _f32),
            jax.ShapeDtypeStruct((E, D), _f32),
        ],
    )(m0, elen, shw, alpha16, sg, gmax, wlen, wtp3, we3,
      ws1, bs1, ws2, bs2, ws3, bs3, wout3)


def _scatter_call(y, dst):
    @functools.partial(
        pl.kernel, mesh=_sc_mesh(),
        out_type=jax.ShapeDtypeStruct((N, D), _f32),
        scratch_types=[
            pltpu.VMEM_SHARED((NPAD, D), _f32),
            pltpu.VMEM((ZR2, D), _f32),
            pltpu.VMEM((CH, D), _f32),
            pltpu.VMEM((CH,), jnp.int32),
        ],
    )
    def k(y_hbm, dst_hbm, out_hbm, shared, zbuf, ybuf, idx):
        cid = lax.axis_index("c")
        sid = lax.axis_index("s")
        base = sid * EPS
        lo = cid * NHALF

        @pl.loop(0, ZR2, unroll=2)
        def _(r):
            for k8 in range(D // 16):
                zbuf[r, pl.ds(k8 * 16, 16)] = jnp.zeros((16,), _f32)

        pltpu.sync_copy(zbuf, shared.at[pl.ds(sid * ZR2, ZR2)])
        plsc.subcore_barrier()

        def chunk(off, ndup):
            pltpu.sync_copy(dst_hbm.at[pl.ds(off, CH)], idx)
            pltpu.sync_copy(y_hbm.at[pl.ds(off, CH)], ybuf)

            @pl.loop(0, CH // 16, unroll=8)
            def _(t):
                sl = pl.ds(t * 16, 16)
                dv = idx[sl] - lo
                inr = (dv >= 0) & (dv < NHALF)
                idx[sl] = jnp.where(inr, dv, NHALF)

            if ndup:
                @pl.loop(0, ndup)
                def _(r):
                    for k8 in range(D // 16):
                        ybuf[r, pl.ds(k8 * 16, 16)] = jnp.zeros((16,), _f32)

            pltpu.sync_copy(ybuf, shared.at[idx], add=True)

        @pl.loop(0, SFULL)
        def _(j):
            chunk(pl.multiple_of(base + j * CH, 8), 0)

        chunk(base + STOFF, STDUP)
        plsc.subcore_barrier()

        @pl.when(sid == 0)
        def _():
            pltpu.sync_copy(shared.at[pl.ds(0, NHALF)],
                            out_hbm.at[pl.ds(lo, NHALF)])

    return k(y, dst)


def kernel(node_in, node_embed, edge_sh, edge_length_embedding, W_pre,
           W_r1, b_r1, W_r2, b_r2, W_tp, W_a1, b_a1, W_a2, b_a2, W_a3,
           b_a3, W_e, W_s1, b_s1, W_s2, b_s2, W_s3, b_s3, W_out,
           edge_src, edge_dst, batch):
    elen = edge_length_embedding
    wsrc = W_pre[0:D]
    wdst = W_pre[D:2 * D]
    wlen = W_pre[2 * D:]
    wtp3 = (W_tp.reshape(D, DSH, DH).transpose(1, 0, 2)
            .reshape(DSH * D, DH).astype(jnp.bfloat16))
    we3 = W_e.reshape(H, D, 64).astype(jnp.bfloat16)
    wout3 = W_out.reshape(H, D, D).astype(jnp.bfloat16)
    wa3p = jnp.pad(W_a3, ((0, 0), (0, 16 - H)))
    ba3p = jnp.pad(b_a3, (0, 16 - H)).reshape(1, 16)
    row = lambda b: b.reshape(1, -1)

    p_tab, q_tab = _pre_call(node_in, wsrc, wdst)
    m0 = _gather_call(p_tab, q_tab, edge_src, edge_dst)
    alpha16, shw, gmax = _alpha_call(
        elen, edge_sh, W_r1, row(b_r1), W_r2, row(b_r2),
        W_a1, row(b_a1), W_a2, row(b_a2), wa3p, ba3p)
    gm16 = jnp.broadcast_to(gmax.reshape(()), (16,))
    ssum = _ssum_call(alpha16, gm16, edge_dst)
    sg = _sgather_call(ssum, edge_dst)
    edge_scalar, y = _main_call(
        m0, elen, shw, alpha16, sg, gmax, wlen, wtp3, we3,
        W_s1, row(b_s1), W_s2, row(b_s2), W_s3, row(b_s3), wout3)
    node_out = _scatter_call(y, edge_dst)
    return (node_out, edge_scalar)


# speedup vs baseline: 1.0056x; 1.0056x over previous
import functools

import jax
import jax.numpy as jnp
from jax import lax
from jax.experimental import pallas as pl
from jax.experimental.pallas import tpu as pltpu
from jax.experimental.pallas import tpu_sc as plsc

N = 10000
E = 160000
D = 128
H = 4
DSH = 4
DLEN = 32
DH = D * H

NC = 2
NS = 16
NW = NC * NS
EPT = E // NW
CH = 128
NFULL = EPT // CH
TOFF = EPT - CH
TDUP = NFULL * CH - TOFF
NHALF = N // NC
NPAD = 5008
EPS = E // NS
SFULL = EPS // CH
STOFF = EPS - CH
STDUP = SFULL * CH - STOFF
ZR2 = NPAD // NS

_f32 = jnp.float32


def _ln(x, eps=1e-6):
    mu = jnp.mean(x, axis=-1, keepdims=True)
    var = jnp.var(x, axis=-1, keepdims=True)
    return (x - mu) / jnp.sqrt(var + eps)


def _silu(x):
    return x * jax.nn.sigmoid(x)


_BN = 2000


def _pre_body(node_ref, wsrc_ref, wdst_ref, p_ref, q_ref):
    x = node_ref[...]
    p_ref[...] = jnp.dot(x, wsrc_ref[...], preferred_element_type=_f32)
    q_ref[...] = jnp.dot(x, wdst_ref[...], preferred_element_type=_f32)


def _pre_call(node_in, wsrc, wdst):
    g = N // _BN
    return pl.pallas_call(
        _pre_body,
        grid=(g,),
        in_specs=[
            pl.BlockSpec((_BN, D), lambda i: (i, 0)),
            pl.BlockSpec((D, D), lambda i: (0, 0)),
            pl.BlockSpec((D, D), lambda i: (0, 0)),
        ],
        out_specs=[
            pl.BlockSpec((_BN, D), lambda i: (i, 0)),
            pl.BlockSpec((_BN, D), lambda i: (i, 0)),
        ],
        out_shape=[
            jax.ShapeDtypeStruct((N, D), _f32),
            jax.ShapeDtypeStruct((N, D), _f32),
        ],
    )(node_in, wsrc, wdst)


_BA = 8000


def _alpha_body(elen_ref, sh_ref, wr1_ref, br1_ref, wr2_ref, br2_ref,
                wa1_ref, ba1_ref, wa2_ref, ba2_ref, wa3_ref, ba3_ref,
                alpha_ref, shw_ref, gmax_ref):
    el = elen_ref[...]
    r = jnp.dot(_silu(jnp.dot(el, wr1_ref[...], preferred_element_type=_f32)
                      + br1_ref[...]),
                wr2_ref[...], preferred_element_type=_f32) + br2_ref[...]
    shw_ref[...] = sh_ref[...] * r
    h1 = _silu(_ln(jnp.dot(el, wa1_ref[...], preferred_element_type=_f32)
                   + ba1_ref[...]))
    h2 = _silu(_ln(jnp.dot(h1, wa2_ref[...], preferred_element_type=_f32)
                   + ba2_ref[...]))
    alpha16 = jnp.dot(h2, wa3_ref[...], preferred_element_type=_f32) + ba3_ref[...]
    alpha_ref[...] = alpha16
    m = jnp.max(alpha16)
    i = pl.program_id(0)

    @pl.when(i == 0)
    def _():
        gmax_ref[0, 0] = m

    @pl.when(i > 0)
    def _():
        gmax_ref[0, 0] = jnp.maximum(gmax_ref[0, 0], m)


def _alpha_call(elen, edge_sh, wr1, br1, wr2, br2, wa1, ba1, wa2, ba2,
                wa3p, ba3p):
    g = E // _BA
    full = lambda r, c: pl.BlockSpec((r, c), lambda i: (0, 0))
    return pl.pallas_call(
        _alpha_body,
        grid=(g,),
        in_specs=[
            pl.BlockSpec((_BA, DLEN), lambda i: (i, 0)),
            pl.BlockSpec((_BA, DSH), lambda i: (i, 0)),
            full(DLEN, 64), full(1, 64), full(64, DSH), full(1, DSH),
            full(DLEN, 64), full(1, 64), full(64, 64), full(1, 64),
            full(64, 16), full(1, 16),
        ],
        out_specs=[
            pl.BlockSpec((_BA, 16), lambda i: (i, 0)),
            pl.BlockSpec((_BA, DSH), lambda i: (i, 0)),
            pl.BlockSpec(memory_space=pltpu.SMEM),
        ],
        out_shape=[
            jax.ShapeDtypeStruct((E, 16), _f32),
            jax.ShapeDtypeStruct((E, DSH), _f32),
            jax.ShapeDtypeStruct((1, 1), _f32),
        ],
    )(elen, edge_sh, wr1, br1, wr2, br2, wa1, ba1, wa2, ba2, wa3p, ba3p)


def _sc_mesh():
    return plsc.VectorSubcoreMesh(core_axis_name="c", subcore_axis_name="s",
                                  num_cores=NC, num_subcores=NS)


def _ssum_call(alpha16, gm16, dst):
    @functools.partial(
        pl.kernel, mesh=_sc_mesh(),
        out_type=jax.ShapeDtypeStruct((N, 16), _f32),
        compiler_params=pltpu.CompilerParams(use_tc_tiling_on_sc=False),
        scratch_types=[
            pltpu.VMEM_SHARED((NPAD, 16), _f32),
            pltpu.VMEM((ZR2, 16), _f32),
            pltpu.VMEM((CH, 16), _f32),
            pltpu.VMEM((CH,), jnp.int32),
            pltpu.VMEM((16,), _f32),
        ],
    )
    def k(alpha_hbm, gm_hbm, dst_hbm, out_hbm, shared, zbuf, ebuf, idx, gmv):
        cid = lax.axis_index("c")
        sid = lax.axis_index("s")
        base = sid * EPS
        lo = cid * NHALF

        @pl.loop(0, ZR2, unroll=8)
        def _(r):
            zbuf[r, :] = jnp.zeros((16,), _f32)

        pltpu.sync_copy(zbuf, shared.at[pl.ds(sid * ZR2, ZR2)])
        pltpu.sync_copy(gm_hbm, gmv)
        plsc.subcore_barrier()
        g = gmv[...]

        def chunk(off, ndup):
            pltpu.sync_copy(dst_hbm.at[pl.ds(off, CH)], idx)
            pltpu.sync_copy(alpha_hbm.at[pl.ds(off, CH)], ebuf)

            @pl.loop(0, CH // 16, unroll=8)
            def _(t):
                sl = pl.ds(t * 16, 16)
                dv = idx[sl] - lo
                inr = (dv >= 0) & (dv < NHALF)
                idx[sl] = jnp.where(inr, dv, NHALF)

            @pl.loop(0, CH, unroll=8)
            def _(r):
                ebuf[r, :] = jnp.exp(ebuf[r, :] - g)

            if ndup:
                @pl.loop(0, ndup)
                def _(r):
                    ebuf[r, :] = jnp.zeros((16,), _f32)

            pltpu.sync_copy(ebuf, shared.at[idx], add=True)

        @pl.loop(0, SFULL)
        def _(j):
            chunk(pl.multiple_of(base + j * CH, 8), 0)

        chunk(base + STOFF, STDUP)
        plsc.subcore_barrier()

        @pl.when(sid == 0)
        def _():
            pltpu.sync_copy(shared.at[pl.ds(0, NHALF)],
                            out_hbm.at[pl.ds(lo, NHALF)])

    return k(alpha16, gm16, dst)


def _gather_call(p_tab, q_tab, src, dst):
    @functools.partial(
        pl.kernel, mesh=_sc_mesh(),
        out_type=jax.ShapeDtypeStruct((E, D), _f32),
        scratch_types=[
            pltpu.VMEM((CH,), jnp.int32),
            pltpu.VMEM((CH,), jnp.int32),
            pltpu.VMEM((CH, D), _f32),
            pltpu.VMEM((CH, D), _f32),
            pltpu.VMEM((CH,), jnp.int32),
            pltpu.VMEM((CH,), jnp.int32),
            pltpu.VMEM((CH, D), _f32),
            pltpu.VMEM((CH, D), _f32),
            pltpu.SemaphoreType.DMA,
            pltpu.SemaphoreType.DMA,
            pltpu.SemaphoreType.DMA,
            pltpu.SemaphoreType.DMA,
            pltpu.SemaphoreType.DMA,
            pltpu.SemaphoreType.DMA,
        ],
    )
    def k(p_hbm, q_hbm, src_hbm, dst_hbm, m0_hbm,
          idxs0, idxd0, bufp0, bufq0, idxs1, idxd1, bufp1, bufq1,
          si0, si1, sg0, sg1, sw0, sw1):
        cid = lax.axis_index("c")
        sid = lax.axis_index("s")
        wid = sid * NC + cid
        base = wid * EPT
        sets = ((idxs0, idxd0, bufp0, bufq0, si0, sg0, sw0),
                (idxs1, idxd1, bufp1, bufq1, si1, sg1, sw1))

        def off_of(j):
            return pl.multiple_of(
                jnp.where(j == NFULL, base + TOFF, base + j * CH), 8)

        @pl.loop(0, (NFULL + 1) // 2)
        def _(t):
            descs = []
            for u, (ixs, ixd, bp, bq, si, sg, sw) in enumerate(sets):
                off = off_of(2 * t + u)
                descs.append((
                    pltpu.async_copy(src_hbm.at[pl.ds(off, CH)], ixs, si),
                    pltpu.async_copy(dst_hbm.at[pl.ds(off, CH)], ixd, si),
                    off))
            gds = []
            for u, (ixs, ixd, bp, bq, si, sg, sw) in enumerate(sets):
                descs[u][0].wait()
                descs[u][1].wait()
                gds.append((pltpu.async_copy(p_hbm.at[ixs], bp, sg),
                            pltpu.async_copy(q_hbm.at[ixd], bq, sg)))
            wbs = []
            for u, (ixs, ixd, bp, bq, si, sg, sw) in enumerate(sets):
                gds[u][0].wait()
                gds[u][1].wait()

                @pl.loop(0, CH, unroll=4)
                def _(r):
                    for k8 in range(D // 16):
                        sl = pl.ds(k8 * 16, 16)
                        bp[r, sl] = bp[r, sl] + bq[r, sl]

                wbs.append(pltpu.async_copy(
                    bp, m0_hbm.at[pl.ds(descs[u][2], CH)], sw))
            for u in range(2):
                wbs[u].wait()

    return k(p_tab, q_tab, src, dst)


def _sgather_call(ssum, dst):
    @functools.partial(
        pl.kernel, mesh=_sc_mesh(),
        out_type=jax.ShapeDtypeStruct((E, 16), _f32),
        compiler_params=pltpu.CompilerParams(use_tc_tiling_on_sc=False),
        scratch_types=[
            pltpu.VMEM((CH,), jnp.int32),
            pltpu.VMEM((CH, 16), _f32),
        ],
    )
    def k(ss_hbm, dst_hbm, sg_hbm, idxd, bufa):
        cid = lax.axis_index("c")
        sid = lax.axis_index("s")
        wid = sid * NC + cid
        base = wid * EPT

        def chunk(off):
            pltpu.sync_copy(dst_hbm.at[pl.ds(off, CH)], idxd)
            pltpu.sync_copy(ss_hbm.at[idxd], bufa)
            pltpu.sync_copy(bufa, sg_hbm.at[pl.ds(off, CH)])

        @pl.loop(0, NFULL)
        def _(j):
            chunk(pl.multiple_of(base + j * CH, 8))

        chunk(base + TOFF)

    return k(ssum, dst)


_BC = 1280


def _main_body(m0_ref, elen_ref, shw_ref, alpha_ref, sg_ref, gmax_ref,
               wlen_ref, wtp_ref, we_ref, ws1_ref, bs1_ref, ws2_ref,
               bs2_ref, ws3_ref, bs3_ref, wout_ref, es_ref, y_ref):
    el = elen_ref[...]
    msg = m0_ref[...] + jnp.dot(el, wlen_ref[...], preferred_element_type=_f32)
    shw = shw_ref[...].astype(jnp.bfloat16)
    msg_b = msg.astype(jnp.bfloat16)
    feat = jnp.concatenate(
        [msg_b * shw[:, s:s + 1] for s in range(DSH)], axis=1)
    value = jnp.dot(feat, wtp_ref[...],
                    preferred_element_type=_f32).astype(jnp.bfloat16)
    alpha16 = alpha_ref[...]
    es = None
    for h in range(H):
        t = jnp.dot(value[:, h * D:(h + 1) * D], we_ref[h],
                    preferred_element_type=_f32) * alpha16[:, h:h + 1]
        es = t if es is None else es + t
    es = _silu(_ln(jnp.dot(es, ws1_ref[...], preferred_element_type=_f32)
                   + bs1_ref[...]))
    es = _silu(_ln(jnp.dot(es, ws2_ref[...], preferred_element_type=_f32)
                   + bs2_ref[...]))
    es_ref[...] = jnp.dot(es, ws3_ref[...], preferred_element_type=_f32) \
        + bs3_ref[...]
    exa = jnp.exp(alpha16 - gmax_ref[0, 0])
    a = exa / (sg_ref[...] + 1e-16)
    y = None
    for h in range(H):
        t = jnp.dot(value[:, h * D:(h + 1) * D], wout_ref[h],
                    preferred_element_type=_f32) * a[:, h:h + 1]
        y = t if y is None else y + t
    y_ref[...] = y


def _main_call(m0, elen, shw, alpha16, sg, gmax, wlen, wtp3, we3,
               ws1, bs1, ws2, bs2, ws3, bs3, wout3):
    g = E // _BC
    full = lambda *s: pl.BlockSpec(s, lambda i: (0,) * len(s))
    return pl.pallas_call(
        _main_body,
        grid=(g,),
        in_specs=[
            pl.BlockSpec((_BC, D), lambda i: (i, 0)),
            pl.BlockSpec((_BC, DLEN), lambda i: (i, 0)),
            pl.BlockSpec((_BC, DSH), lambda i: (i, 0)),
            pl.BlockSpec((_BC, 16), lambda i: (i, 0)),
            pl.BlockSpec((_BC, 16), lambda i: (i, 0)),
            pl.BlockSpec(memory_space=pltpu.SMEM),
            full(DLEN, D),
            full(DSH * D, DH),
            full(H, D, 64),
            full(64, 64), full(1, 64),
            full(64, 64), full(1, 64),
            full(64, 32), full(1, 32),
            full(H, D, D),
        ],
        out_specs=[
            pl.BlockSpec((_BC, 32), lambda i: (i, 0)),
            pl.BlockSpec((_BC, D), lambda i: (i, 0)),
        ],
        out_shape=[
            jax.ShapeDtypeStruct((E, 32), _f32),
            jax.ShapeDtypeStruct((E, D), _f32),
        ],
    )(m0, elen, shw, alpha16, sg, gmax, wlen, wtp3, we3,
      ws1, bs1, ws2, bs2, ws3, bs3, wout3)


def _scatter_call(y, dst):
    @functools.partial(
        pl.kernel, mesh=_sc_mesh(),
        out_type=jax.ShapeDtypeStruct((N, D), _f32),
        scratch_types=[
            pltpu.VMEM_SHARED((NPAD, D), _f32),
            pltpu.VMEM((ZR2, D), _f32),
            pltpu.VMEM((CH, D), _f32),
            pltpu.VMEM((CH,), jnp.int32),
        ],
    )
    def k(y_hbm, dst_hbm, out_hbm, shared, zbuf, ybuf, idx):
        cid = lax.axis_index("c")
        sid = lax.axis_index("s")
        base = sid * EPS
        lo = cid * NHALF

        @pl.loop(0, ZR2, unroll=2)
        def _(r):
            for k8 in range(D // 16):
                zbuf[r, pl.ds(k8 * 16, 16)] = jnp.zeros((16,), _f32)

        pltpu.sync_copy(zbuf, shared.at[pl.ds(sid * ZR2, ZR2)])
        plsc.subcore_barrier()

        def chunk(off, ndup):
            pltpu.sync_copy(dst_hbm.at[pl.ds(off, CH)], idx)
            pltpu.sync_copy(y_hbm.at[pl.ds(off, CH)], ybuf)

            @pl.loop(0, CH // 16, unroll=8)
            def _(t):
                sl = pl.ds(t * 16, 16)
                dv = idx[sl] - lo
                inr = (dv >= 0) & (dv < NHALF)
                idx[sl] = jnp.where(inr, dv, NHALF)

            if ndup:
                @pl.loop(0, ndup)
                def _(r):
                    for k8 in range(D // 16):
                        ybuf[r, pl.ds(k8 * 16, 16)] = jnp.zeros((16,), _f32)

            pltpu.sync_copy(ybuf, shared.at[idx], add=True)

        @pl.loop(0, SFULL)
        def _(j):
            chunk(pl.multiple_of(base + j * CH, 8), 0)

        chunk(base + STOFF, STDUP)
        plsc.subcore_barrier()

        @pl.when(sid == 0)
        def _():
            pltpu.sync_copy(shared.at[pl.ds(0, NHALF)],
                            out_hbm.at[pl.ds(lo, NHALF)])

    return k(y, dst)


def kernel(node_in, node_embed, edge_sh, edge_length_embedding, W_pre,
           W_r1, b_r1, W_r2, b_r2, W_tp, W_a1, b_a1, W_a2, b_a2, W_a3,
           b_a3, W_e, W_s1, b_s1, W_s2, b_s2, W_s3, b_s3, W_out,
           edge_src, edge_dst, batch):
    elen = edge_length_embedding
    wsrc = W_pre[0:D]
    wdst = W_pre[D:2 * D]
    wlen = W_pre[2 * D:]
    wtp3 = (W_tp.reshape(D, DSH, DH).transpose(1, 0, 2)
            .reshape(DSH * D, DH).astype(jnp.bfloat16))
    we3 = W_e.reshape(H, D, 64).astype(jnp.bfloat16)
    wout3 = W_out.reshape(H, D, D).astype(jnp.bfloat16)
    wa3p = jnp.pad(W_a3, ((0, 0), (0, 16 - H)))
    ba3p = jnp.pad(b_a3, (0, 16 - H)).reshape(1, 16)
    row = lambda b: b.reshape(1, -1)

    p_tab, q_tab = _pre_call(node_in, wsrc, wdst)
    m0 = _gather_call(p_tab, q_tab, edge_src, edge_dst)
    alpha16, shw, gmax = _alpha_call(
        elen, edge_sh, W_r1, row(b_r1), W_r2, row(b_r2),
        W_a1, row(b_a1), W_a2, row(b_a2), wa3p, ba3p)
    gm16 = jnp.broadcast_to(gmax.reshape(()), (16,))
    ssum = _ssum_call(alpha16, gm16, edge_dst)
    sg = _sgather_call(ssum, edge_dst)
    edge_scalar, y = _main_call(
        m0, elen, shw, alpha16, sg, gmax, wlen, wtp3, we3,
        W_s1, row(b_s1), W_s2, row(b_s2), W_s3, row(b_s3), wout3)
    node_out = _scatter_call(y, edge_dst)
    return (node_out, edge_scalar)
